# Initial kernel scaffold; baseline (speedup 1.0000x reference)
import jax
import jax.numpy as jnp
from jax.experimental import pallas as pl


def kernel(node_features, edge_index, edge_features, global_features, ed_w1, ed_b1, ed_w2, ed_b2, nd_w1, nd_b1, nd_w2, nd_b2, gd_w1, gd_b1, gd_w2, gd_b2, em_w, em_b, nm_w, nm_b, gm_w, gm_b):
    raise NotImplementedError("write your pallas kernel here")


# speedup vs baseline: 4.1901x; 4.1901x over previous
import functools
import jax
import jax.numpy as jnp
from jax import lax
from jax.experimental import pallas as pl
from jax.experimental.pallas import tpu as pltpu
from jax.experimental.pallas import tpu_sc as plsc

N2 = 128
H = 64
LANES = 16
GW = H + LANES


def _sp2(x):
    return jnp.maximum(x, 0.0) + jnp.log1p(jnp.exp(-jnp.abs(x))) - jnp.log(2.0).astype(x.dtype)


def _edge_mlp_body(ef_ref, w1_ref, b1_ref, w2_ref, b2_ref, wc_ref, cst_ref, out_ref):
    h = _sp2(jnp.dot(ef_ref[...], w1_ref[...], preferred_element_type=jnp.float32) + b1_ref[...])
    e = _sp2(jnp.dot(h, w2_ref[...], preferred_element_type=jnp.float32) + b2_ref[...])
    q = jnp.dot(e, wc_ref[...], preferred_element_type=jnp.float32) + cst_ref[...]
    out_ref[0] = q[:, :H]
    out_ref[1] = q[:, H:]


def _edge_mlp(ef0, w1, b1, w2, b2, wc, cst, block):
    E = ef0.shape[0]
    n1 = w1.shape[1]
    grid = (E // block,)
    return pl.pallas_call(
        _edge_mlp_body,
        grid=grid,
        in_specs=[
            pl.BlockSpec((block, N2), lambda i: (i, 0)),
            pl.BlockSpec((N2, n1), lambda i: (0, 0)),
            pl.BlockSpec((1, n1), lambda i: (0, 0)),
            pl.BlockSpec((n1, N2), lambda i: (0, 0)),
            pl.BlockSpec((1, N2), lambda i: (0, 0)),
            pl.BlockSpec((N2, N2), lambda i: (0, 0)),
            pl.BlockSpec((1, N2), lambda i: (0, 0)),
        ],
        out_specs=pl.BlockSpec((2, block, H), lambda i: (0, i, 0)),
        out_shape=jax.ShapeDtypeStruct((2, E, H), jnp.float32),
    )(ef0, w1, b1, w2, b2, wc, cst)


def _node_mlp_body(nf_ref, w1_ref, b1_ref, w2_ref, b2_ref, wa_ref, wb_ref,
                   nf_out_ref, pa_ref, pb_ref):
    h = _sp2(jnp.dot(nf_ref[...], w1_ref[...], preferred_element_type=jnp.float32) + b1_ref[...])
    nf = _sp2(jnp.dot(h, w2_ref[...], preferred_element_type=jnp.float32) + b2_ref[...])
    nf_out_ref[...] = nf
    pa = jnp.dot(nf, wa_ref[...], preferred_element_type=jnp.float32)
    pb = jnp.dot(nf, wb_ref[...], preferred_element_type=jnp.float32)
    pa_ref[0] = pa[:, :H]
    pa_ref[1] = pa[:, H:]
    pb_ref[0] = pb[:, :H]
    pb_ref[1] = pb[:, H:]


def _node_mlp(nf0, w1, b1, w2, b2, wa, wb, block):
    N = nf0.shape[0]
    n1 = w1.shape[1]
    grid = (N // block,)
    hblk = pl.BlockSpec((2, block, H), lambda i: (0, i, 0))
    return pl.pallas_call(
        _node_mlp_body,
        grid=grid,
        in_specs=[
            pl.BlockSpec((block, N2), lambda i: (i, 0)),
            pl.BlockSpec((N2, n1), lambda i: (0, 0)),
            pl.BlockSpec((1, n1), lambda i: (0, 0)),
            pl.BlockSpec((n1, N2), lambda i: (0, 0)),
            pl.BlockSpec((1, N2), lambda i: (0, 0)),
            pl.BlockSpec((N2, N2), lambda i: (0, 0)),
            pl.BlockSpec((N2, N2), lambda i: (0, 0)),
        ],
        out_specs=[pl.BlockSpec((block, N2), lambda i: (i, 0)), hblk, hblk],
        out_shape=[jax.ShapeDtypeStruct((N, N2), jnp.float32),
                   jax.ShapeDtypeStruct((2, N, H), jnp.float32),
                   jax.ShapeDtypeStruct((2, N, H), jnp.float32)],
    )(nf0, w1, b1, w2, b2, wa, wb)


def _sc_edge_kernel(N, E, sidx, didx, pa2, pb2, qc2, ef0v):
    NC, NS = 2, 16
    EPT = E // NS
    C = 80
    NCHUNK = EPT // C
    ZR = 80
    NRB = N // ZR
    mesh = plsc.VectorSubcoreMesh(core_axis_name="c", subcore_axis_name="s",
                                  num_cores=NC, num_subcores=NS)

    @functools.partial(
        pl.kernel,
        mesh=mesh,
        out_type=[
            jax.ShapeDtypeStruct((E, 2, H), jnp.float32),
            jax.ShapeDtypeStruct((NC, N, GW), jnp.float32),
        ],
        scratch_types=[
            pltpu.VMEM((C,), jnp.int32),
            pltpu.VMEM((C,), jnp.int32),
            pltpu.VMEM((C,), jnp.int32),
            pltpu.VMEM((C,), jnp.int32),
            pltpu.VMEM((C, H), jnp.float32),
            pltpu.VMEM((C, H), jnp.float32),
            pltpu.VMEM((C, H), jnp.float32),
            pltpu.VMEM((C, H), jnp.float32),
            pltpu.VMEM((C, H), jnp.float32),
            pltpu.VMEM((C, 1, H), jnp.float32),
            pltpu.VMEM((C, 1, H), jnp.float32),
            pltpu.VMEM((C, GW), jnp.float32),
            pltpu.VMEM((C, GW), jnp.float32),
            pltpu.VMEM((ZR, GW), jnp.float32),
            pltpu.VMEM_SHARED((N, GW), jnp.float32),
            pltpu.SemaphoreType.DMA,
        ],
        compiler_params=pltpu.CompilerParams(use_tc_tiling_on_sc=False),
    )
    def k(sidx_h, didx_h, pa_h, pb_h, qc_h, ef0_h, efout_h, gout_h,
          sbuf, dbuf, sadj, dadj, pa_s, pa_d, pb_s, pb_d, qcv, ef0b, efo,
          fwd, bwd, zbuf, gacc, sem):
        cid = lax.axis_index("c")
        sid = lax.axis_index("s")

        zero = jnp.zeros((LANES,), jnp.float32)
        one0 = jnp.where(lax.iota(jnp.int32, LANES) == 0,
                         jnp.float32(1.0), jnp.float32(0.0))
        roff = jnp.broadcast_to(cid * N, (LANES,)).astype(jnp.int32)

        def zb(r, _):
            for cc in range(GW // LANES):
                zbuf[r, pl.ds(cc * LANES, LANES)] = zero
            return 0
        lax.fori_loop(0, ZR, zb, 0)

        for kk in range((NRB + NS - 1) // NS):
            b = sid + kk * NS

            @pl.when(b < NRB)
            def _():
                pltpu.sync_copy(zbuf, gacc.at[pl.ds(b * ZR, ZR)])

        def init_pay(r, _):
            fwd[r, pl.ds(H, LANES)] = one0
            bwd[r, pl.ds(H, LANES)] = one0
            return 0
        lax.fori_loop(0, C, init_pay, 0)

        plsc.subcore_barrier()

        def chunk(ci, _):
            base = sid * EPT + ci * C
            pltpu.sync_copy(sidx_h.at[pl.ds(base, C)], sbuf)
            pltpu.sync_copy(didx_h.at[pl.ds(base, C)], dbuf)

            def adj(j, _):
                sl = pl.ds(j * LANES, LANES)
                sadj[sl] = sbuf[sl] + roff
                dadj[sl] = dbuf[sl] + roff
                return 0
            lax.fori_loop(0, C // LANES, adj, 0)

            cps = [
                pltpu.async_copy(pa_h.at[sadj], pa_s, sem),
                pltpu.async_copy(pa_h.at[dadj], pa_d, sem),
                pltpu.async_copy(pb_h.at[sadj], pb_s, sem),
                pltpu.async_copy(pb_h.at[dadj], pb_d, sem),
                pltpu.async_copy(qc_h.at[pl.ds(cid * E + base, C)], qcv, sem),
                pltpu.async_copy(ef0_h.at[pl.ds(base, C), pl.ds(cid, 1)], ef0b, sem),
            ]
            for cp in cps:
                cp.wait()

            def row(r, _):
                for cc in range(H // LANES):
                    sl = pl.ds(cc * LANES, LANES)
                    q = qcv[r, sl]
                    f = pa_s[r, sl] + pb_d[r, sl] + q
                    b = pa_d[r, sl] + pb_s[r, sl] + q
                    fwd[r, sl] = f
                    bwd[r, sl] = b
                    efo[r, 0, sl] = (f + b) * 0.5 + ef0b[r, 0, sl]
                return 0
            lax.fori_loop(0, C, row, 0)

            pltpu.sync_copy(fwd, gacc.at[dbuf], add=True)
            pltpu.sync_copy(bwd, gacc.at[sbuf], add=True)
            pltpu.sync_copy(efo, efout_h.at[pl.ds(base, C), pl.ds(cid, 1)])
            return 0
        lax.fori_loop(0, NCHUNK, chunk, 0)

        plsc.subcore_barrier()

        for kk in range((NRB + NS - 1) // NS):
            b = sid + kk * NS

            @pl.when(b < NRB)
            def _():
                pltpu.sync_copy(gacc.at[pl.ds(b * ZR, ZR)],
                                gout_h.at[cid, pl.ds(b * ZR, ZR)])

    return k(sidx, didx, pa2, pb2, qc2, ef0v)


def _node_update_body(g0_ref, g1_ref, nf_ref, nf0_ref, wn1_ref, wn2_ref, cstn_ref,
                      out_ref, accn_ref, accg_ref):
    g0 = g0_ref[0]
    g1 = g1_ref[0]
    sums = jnp.concatenate([g0[:, :H], g1[:, :H]], axis=1)
    deg = g0[:, H:H + 1]
    efm = sums / jnp.maximum(deg, 1.0)
    nf_new = (jnp.dot(nf_ref[...], wn1_ref[...], preferred_element_type=jnp.float32)
              + jnp.dot(efm, wn2_ref[...], preferred_element_type=jnp.float32)
              + cstn_ref[...])
    out_ref[...] = nf_new + nf0_ref[...]

    @pl.when(pl.program_id(0) == 0)
    def _():
        accn_ref[...] = jnp.zeros_like(accn_ref)
        accg_ref[...] = jnp.zeros_like(accg_ref)
    accn_ref[...] += jnp.sum(nf_new, axis=0, keepdims=True)
    accg_ref[...] += jnp.sum(sums, axis=0, keepdims=True)


def _node_update(gparts, nf, nf0, wn1, wn2, cstn, block):
    N = nf.shape[0]
    grid = (N // block,)
    blk = pl.BlockSpec((block, N2), lambda i: (i, 0))
    gblk0 = pl.BlockSpec((1, block, GW), lambda i: (0, i, 0))
    gblk1 = pl.BlockSpec((1, block, GW), lambda i: (1, i, 0))
    return pl.pallas_call(
        _node_update_body,
        grid=grid,
        in_specs=[
            gblk0, gblk1, blk, blk,
            pl.BlockSpec((N2, N2), lambda i: (0, 0)),
            pl.BlockSpec((N2, N2), lambda i: (0, 0)),
            pl.BlockSpec((1, N2), lambda i: (0, 0)),
        ],
        out_specs=[blk,
                   pl.BlockSpec((1, N2), lambda i: (0, 0)),
                   pl.BlockSpec((1, N2), lambda i: (0, 0))],
        out_shape=[jax.ShapeDtypeStruct((N, N2), jnp.float32),
                   jax.ShapeDtypeStruct((1, N2), jnp.float32),
                   jax.ShapeDtypeStruct((1, N2), jnp.float32)],
    )(gparts, gparts, nf, nf0, wn1, wn2, cstn)


def kernel(node_features, edge_index, edge_features, global_features,
           ed_w1, ed_b1, ed_w2, ed_b2,
           nd_w1, nd_b1, nd_w2, nd_b2,
           gd_w1, gd_b1, gd_w2, gd_b2,
           em_w, em_b, nm_w, nm_b, gm_w, gm_b):
    N = node_features.shape[0]
    E = edge_features.shape[0]
    nf0, ef0, gf0 = node_features, edge_features, global_features

    Wa, Wb, Wc, Wd = em_w[:N2], em_w[N2:2 * N2], em_w[2 * N2:3 * N2], em_w[3 * N2:]
    Wn1, Wn2, Wn3 = nm_w[:N2], nm_w[N2:2 * N2], nm_w[2 * N2:]
    Wg1, Wg2, Wg3 = gm_w[:N2], gm_w[N2:2 * N2], gm_w[2 * N2:]

    gf = _sp2(_sp2(gf0 @ gd_w1 + gd_b1) @ gd_w2 + gd_b2)
    cst_e = gf @ Wd + em_b.reshape(1, N2)
    cst_n = gf @ Wn3 + nm_b.reshape(1, N2)

    qc2 = _edge_mlp(ef0, ed_w1, ed_b1.reshape(1, -1), ed_w2, ed_b2.reshape(1, -1),
                    Wc, cst_e, block=2000)
    nf, pa2, pb2 = _node_mlp(nf0, nd_w1, nd_b1.reshape(1, -1), nd_w2,
                             nd_b2.reshape(1, -1), Wa, Wb, block=2000)

    ef_out3, gparts = _sc_edge_kernel(
        N, E, edge_index[0], edge_index[1],
        pa2.reshape(2 * N, H), pb2.reshape(2 * N, H),
        qc2.reshape(2 * E, H), ef0.reshape(E, 2, H))
    ef_out = ef_out3.reshape(E, N2)

    nf_out, sum_nfnew, sum_g = _node_update(gparts, nf, nf0, Wn1, Wn2, cst_n,
                                            block=2000)

    sum_g = sum_g.reshape(1, N2)
    e_mean = sum_g / (2.0 * E)
    n_mean = sum_nfnew / N
    gf_out = e_mean @ Wg1 + n_mean @ Wg2 + gf @ Wg3 + gm_b.reshape(1, N2) + gf0
    return (nf_out, ef_out, gf_out)


# speedup vs baseline: 9.1028x; 2.1725x over previous
import functools
import jax
import jax.numpy as jnp
from jax import lax
from jax.experimental import pallas as pl
from jax.experimental.pallas import tpu as pltpu
from jax.experimental.pallas import tpu_sc as plsc

N2 = 128
H = 64
LANES = 16
GW = H + LANES


def _sp2(x):
    return jnp.maximum(x, 0.0) + jnp.log1p(jnp.exp(-jnp.abs(x))) - jnp.log(2.0).astype(x.dtype)


def _edge_mlp_body(ef_ref, w1_ref, b1_ref, w2_ref, b2_ref, wc_ref, cst_ref, out_ref):
    h = _sp2(jnp.dot(ef_ref[...], w1_ref[...], preferred_element_type=jnp.float32) + b1_ref[...])
    e = _sp2(jnp.dot(h, w2_ref[...], preferred_element_type=jnp.float32) + b2_ref[...])
    out_ref[...] = jnp.dot(e, wc_ref[...], preferred_element_type=jnp.float32) + cst_ref[...]


def _edge_mlp(ef0, w1, b1, w2, b2, wc, cst, block):
    E = ef0.shape[0]
    n1 = w1.shape[1]
    grid = (E // block,)
    return pl.pallas_call(
        _edge_mlp_body,
        grid=grid,
        in_specs=[
            pl.BlockSpec((block, N2), lambda i: (i, 0)),
            pl.BlockSpec((N2, n1), lambda i: (0, 0)),
            pl.BlockSpec((1, n1), lambda i: (0, 0)),
            pl.BlockSpec((n1, N2), lambda i: (0, 0)),
            pl.BlockSpec((1, N2), lambda i: (0, 0)),
            pl.BlockSpec((N2, N2), lambda i: (0, 0)),
            pl.BlockSpec((1, N2), lambda i: (0, 0)),
        ],
        out_specs=pl.BlockSpec((block, N2), lambda i: (i, 0)),
        out_shape=jax.ShapeDtypeStruct((E, N2), jnp.float32),
    )(ef0, w1, b1, w2, b2, wc, cst)


def _node_mlp_body(nf_ref, w1_ref, b1_ref, w2_ref, b2_ref, wa_ref, wb_ref,
                   nf_out_ref, pa_ref, pb_ref):
    h = _sp2(jnp.dot(nf_ref[...], w1_ref[...], preferred_element_type=jnp.float32) + b1_ref[...])
    nf = _sp2(jnp.dot(h, w2_ref[...], preferred_element_type=jnp.float32) + b2_ref[...])
    nf_out_ref[...] = nf
    pa = jnp.dot(nf, wa_ref[...], preferred_element_type=jnp.float32)
    pb = jnp.dot(nf, wb_ref[...], preferred_element_type=jnp.float32)
    pa_ref[0] = pa[:, :H]
    pa_ref[1] = pa[:, H:]
    pb_ref[0] = pb[:, :H]
    pb_ref[1] = pb[:, H:]


def _node_mlp(nf0, w1, b1, w2, b2, wa, wb, block):
    N = nf0.shape[0]
    n1 = w1.shape[1]
    grid = (N // block,)
    blk = pl.BlockSpec((block, N2), lambda i: (i, 0))
    hblk = pl.BlockSpec((2, block, H), lambda i: (0, i, 0))
    return pl.pallas_call(
        _node_mlp_body,
        grid=grid,
        in_specs=[
            blk,
            pl.BlockSpec((N2, n1), lambda i: (0, 0)),
            pl.BlockSpec((1, n1), lambda i: (0, 0)),
            pl.BlockSpec((n1, N2), lambda i: (0, 0)),
            pl.BlockSpec((1, N2), lambda i: (0, 0)),
            pl.BlockSpec((N2, N2), lambda i: (0, 0)),
            pl.BlockSpec((N2, N2), lambda i: (0, 0)),
        ],
        out_specs=[blk, hblk, hblk],
        out_shape=[jax.ShapeDtypeStruct((N, N2), jnp.float32),
                   jax.ShapeDtypeStruct((2, N, H), jnp.float32),
                   jax.ShapeDtypeStruct((2, N, H), jnp.float32)],
    )(nf0, w1, b1, w2, b2, wa, wb)


def _sc_edge_kernel(N, E, sidx, didx, pa, pb, qc, ef0):
    NC, NS = 2, 16
    EPT = E // NS
    C = 80
    NCHUNK = EPT // C
    ZR = 80
    NRB = N // ZR
    mesh = plsc.VectorSubcoreMesh(core_axis_name="c", subcore_axis_name="s",
                                  num_cores=NC, num_subcores=NS)

    @functools.partial(
        pl.kernel,
        mesh=mesh,
        out_type=[
            jax.ShapeDtypeStruct((E, N2), jnp.float32),
            jax.ShapeDtypeStruct((NC, N, N2), jnp.float32),
        ],
        scratch_types=[
            pltpu.VMEM((C,), jnp.int32),
            pltpu.VMEM((C,), jnp.int32),
            pltpu.VMEM((C,), jnp.int32),
            pltpu.VMEM((C,), jnp.int32),
            pltpu.VMEM((C, H), jnp.float32),
            pltpu.VMEM((C, H), jnp.float32),
            pltpu.VMEM((C, H), jnp.float32),
            pltpu.VMEM((C, H), jnp.float32),
            pltpu.VMEM((C, H), jnp.float32),
            pltpu.VMEM((C, H), jnp.float32),
            pltpu.VMEM((C, H), jnp.float32),
            pltpu.VMEM((C, GW), jnp.float32),
            pltpu.VMEM((C, GW), jnp.float32),
            pltpu.VMEM((ZR, GW), jnp.float32),
            pltpu.VMEM_SHARED((N, GW), jnp.float32),
            pltpu.SemaphoreType.DMA,
        ],
        compiler_params=pltpu.CompilerParams(use_tc_tiling_on_sc=False),
    )
    def k(sidx_h, didx_h, pa_h, pb_h, qc_h, ef0_h, efout_h, gout_h,
          sbuf, dbuf, sadj, dadj, pa_s, pa_d, pb_s, pb_d, qcv, ef0b, efo,
          fwd, bwd, zbuf, gacc, sem):
        cid = lax.axis_index("c")
        sid = lax.axis_index("s")
        col = cid * H
        roff = jnp.broadcast_to(cid * N, (LANES,)).astype(jnp.int32)

        zero = jnp.zeros((LANES,), jnp.float32)
        one0 = jnp.where(lax.iota(jnp.int32, LANES) == 0,
                         jnp.float32(1.0), jnp.float32(0.0))

        def zb(r, _):
            for cc in range(GW // LANES):
                zbuf[r, pl.ds(cc * LANES, LANES)] = zero
            return 0
        lax.fori_loop(0, ZR, zb, 0)

        for kk in range((NRB + NS - 1) // NS):
            b = sid + kk * NS

            @pl.when(b < NRB)
            def _():
                pltpu.sync_copy(zbuf, gacc.at[pl.ds(b * ZR, ZR)])

        def init_pay(r, _):
            fwd[r, pl.ds(H, LANES)] = one0
            bwd[r, pl.ds(H, LANES)] = one0
            return 0
        lax.fori_loop(0, C, init_pay, 0)

        plsc.subcore_barrier()

        def chunk(ci, _):
            base = sid * EPT + ci * C
            pltpu.sync_copy(sidx_h.at[pl.ds(base, C)], sbuf)
            pltpu.sync_copy(didx_h.at[pl.ds(base, C)], dbuf)

            def adj(j, _):
                sl = pl.ds(j * LANES, LANES)
                sadj[sl] = sbuf[sl] + roff
                dadj[sl] = dbuf[sl] + roff
                return 0
            lax.fori_loop(0, C // LANES, adj, 0)

            cps = [
                pltpu.async_copy(pa_h.at[sadj], pa_s, sem),
                pltpu.async_copy(pa_h.at[dadj], pa_d, sem),
                pltpu.async_copy(pb_h.at[sadj], pb_s, sem),
                pltpu.async_copy(pb_h.at[dadj], pb_d, sem),
                pltpu.async_copy(qc_h.at[pl.ds(base, C), pl.ds(col, H)], qcv, sem),
                pltpu.async_copy(ef0_h.at[pl.ds(base, C), pl.ds(col, H)], ef0b, sem),
            ]
            for cp in cps:
                cp.wait()

            def row(r, _):
                for cc in range(H // LANES):
                    sl = pl.ds(cc * LANES, LANES)
                    q = qcv[r, sl]
                    f = pa_s[r, sl] + pb_d[r, sl] + q
                    b = pa_d[r, sl] + pb_s[r, sl] + q
                    fwd[r, sl] = f
                    bwd[r, sl] = b
                    efo[r, sl] = (f + b) * 0.5 + ef0b[r, sl]
                return 0
            lax.fori_loop(0, C, row, 0)

            pltpu.sync_copy(fwd, gacc.at[dbuf], add=True)
            pltpu.sync_copy(bwd, gacc.at[sbuf], add=True)
            pltpu.sync_copy(efo, efout_h.at[pl.ds(base, C), pl.ds(col, H)])
            return 0
        lax.fori_loop(0, NCHUNK, chunk, 0)

        plsc.subcore_barrier()

        for kk in range((NRB + NS - 1) // NS):
            b = sid + kk * NS

            @pl.when(b < NRB)
            def _():
                pltpu.sync_copy(gacc.at[pl.ds(b * ZR, ZR)],
                                gout_h.at[cid, pl.ds(b * ZR, ZR), pl.ds(0, GW)])

    return k(sidx, didx, pa, pb, qc, ef0)


def _node_update_body(g0_ref, g1_ref, nf_ref, nf0_ref, wn1_ref, wn2_ref, cstn_ref,
                      out_ref, accn_ref, accg_ref):
    g0 = g0_ref[0]
    g1 = g1_ref[0]
    sums = jnp.concatenate([g0[:, :H], g1[:, :H]], axis=1)
    deg = g0[:, H:H + 1]
    efm = sums / jnp.maximum(deg, 1.0)
    nf_new = (jnp.dot(nf_ref[...], wn1_ref[...], preferred_element_type=jnp.float32)
              + jnp.dot(efm, wn2_ref[...], preferred_element_type=jnp.float32)
              + cstn_ref[...])
    out_ref[...] = nf_new + nf0_ref[...]

    @pl.when(pl.program_id(0) == 0)
    def _():
        accn_ref[...] = jnp.zeros_like(accn_ref)
        accg_ref[...] = jnp.zeros_like(accg_ref)
    accn_ref[...] += jnp.sum(nf_new, axis=0, keepdims=True)
    accg_ref[...] += jnp.sum(sums, axis=0, keepdims=True)


def _node_update(gparts, nf, nf0, wn1, wn2, cstn, block):
    N = nf.shape[0]
    grid = (N // block,)
    blk = pl.BlockSpec((block, N2), lambda i: (i, 0))
    gblk0 = pl.BlockSpec((1, block, N2), lambda i: (0, i, 0))
    gblk1 = pl.BlockSpec((1, block, N2), lambda i: (1, i, 0))
    return pl.pallas_call(
        _node_update_body,
        grid=grid,
        in_specs=[
            gblk0, gblk1, blk, blk,
            pl.BlockSpec((N2, N2), lambda i: (0, 0)),
            pl.BlockSpec((N2, N2), lambda i: (0, 0)),
            pl.BlockSpec((1, N2), lambda i: (0, 0)),
        ],
        out_specs=[blk,
                   pl.BlockSpec((1, N2), lambda i: (0, 0)),
                   pl.BlockSpec((1, N2), lambda i: (0, 0))],
        out_shape=[jax.ShapeDtypeStruct((N, N2), jnp.float32),
                   jax.ShapeDtypeStruct((1, N2), jnp.float32),
                   jax.ShapeDtypeStruct((1, N2), jnp.float32)],
    )(gparts, gparts, nf, nf0, wn1, wn2, cstn)


def kernel(node_features, edge_index, edge_features, global_features,
           ed_w1, ed_b1, ed_w2, ed_b2,
           nd_w1, nd_b1, nd_w2, nd_b2,
           gd_w1, gd_b1, gd_w2, gd_b2,
           em_w, em_b, nm_w, nm_b, gm_w, gm_b):
    N = node_features.shape[0]
    E = edge_features.shape[0]
    nf0, ef0, gf0 = node_features, edge_features, global_features

    Wa, Wb, Wc, Wd = em_w[:N2], em_w[N2:2 * N2], em_w[2 * N2:3 * N2], em_w[3 * N2:]
    Wn1, Wn2, Wn3 = nm_w[:N2], nm_w[N2:2 * N2], nm_w[2 * N2:]
    Wg1, Wg2, Wg3 = gm_w[:N2], gm_w[N2:2 * N2], gm_w[2 * N2:]

    gf = _sp2(_sp2(gf0 @ gd_w1 + gd_b1) @ gd_w2 + gd_b2)
    cst_e = gf @ Wd + em_b.reshape(1, N2)
    cst_n = gf @ Wn3 + nm_b.reshape(1, N2)

    qc = _edge_mlp(ef0, ed_w1, ed_b1.reshape(1, -1), ed_w2, ed_b2.reshape(1, -1),
                   Wc, cst_e, block=2000)
    nf, pa, pb = _node_mlp(nf0, nd_w1, nd_b1.reshape(1, -1), nd_w2,
                           nd_b2.reshape(1, -1), Wa, Wb, block=2000)

    ef_out, gparts = _sc_edge_kernel(N, E, edge_index[0], edge_index[1],
                                     pa.reshape(2 * N, H), pb.reshape(2 * N, H),
                                     qc, ef0)

    nf_out, sum_nfnew, sum_g = _node_update(gparts, nf, nf0, Wn1, Wn2, cst_n,
                                            block=2000)

    e_mean = sum_g / (2.0 * E)
    n_mean = sum_nfnew / N
    gf_out = e_mean @ Wg1 + n_mean @ Wg2 + gf @ Wg3 + gm_b.reshape(1, N2) + gf0
    return (nf_out, ef_out, gf_out)
